# Initial kernel scaffold; baseline (speedup 1.0000x reference)
import jax
import jax.numpy as jnp
from jax.experimental import pallas as pl


def kernel(x, table):
    raise NotImplementedError("write your pallas kernel here")


# speedup vs baseline: 3.6339x; 3.6339x over previous
import functools

import jax
import jax.numpy as jnp
from jax import lax
from jax.experimental import pallas as pl


def _body(x_ref, t_ref, o_ref, *, bi, s, maxrel):
    i0 = pl.program_id(0) * bi
    nrows = t_ref.shape[0]
    k = lax.broadcasted_iota(jnp.int32, (bi * s, nrows), 0)
    rel = jnp.clip(i0 + k // s - k % s, -maxrel, maxrel) + maxrel
    oh = (rel == lax.broadcasted_iota(jnp.int32, (bi * s, nrows), 1)).astype(jnp.float32)
    emb = jnp.dot(oh, t_ref[...], preferred_element_type=jnp.float32)
    o_ref[...] = x_ref[...] + emb.reshape(bi, s, -1)


@jax.jit
def kernel(x, table):
    s, s2, d = x.shape
    maxrel = (table.shape[0] - 1) // 2
    pad = (-table.shape[0]) % 8
    tpad = jnp.pad(table, ((0, pad), (0, 0)))
    bi = 4
    grid = (s // bi,)
    return pl.pallas_call(
        functools.partial(_body, bi=bi, s=s2, maxrel=maxrel),
        grid=grid,
        in_specs=[
            pl.BlockSpec((bi, s2, d), lambda i: (i, 0, 0)),
            pl.BlockSpec(tpad.shape, lambda i: (0, 0)),
        ],
        out_specs=pl.BlockSpec((bi, s2, d), lambda i: (i, 0, 0)),
        out_shape=jax.ShapeDtypeStruct((s, s2, d), x.dtype),
    )(x, tpad)


# speedup vs baseline: 3.6569x; 1.0063x over previous
import functools

import jax
import jax.numpy as jnp
from jax import lax
from jax.experimental import pallas as pl


def _body(x_ref, t_ref, o_ref, *, bi, s, maxrel):
    i0 = pl.program_id(0) * bi
    nrows = t_ref.shape[0]
    k = lax.broadcasted_iota(jnp.int32, (bi * s, nrows), 0)
    rel = jnp.clip(i0 + k // s - k % s, -maxrel, maxrel) + maxrel
    oh = (rel == lax.broadcasted_iota(jnp.int32, (bi * s, nrows), 1)).astype(jnp.float32)
    emb = jnp.dot(oh, t_ref[...], preferred_element_type=jnp.float32)
    o_ref[...] = x_ref[...] + emb.reshape(bi, s, -1)


@jax.jit
def kernel(x, table):
    s, s2, d = x.shape
    maxrel = (table.shape[0] - 1) // 2
    pad = (-table.shape[0]) % 8
    tpad = jnp.pad(table, ((0, pad), (0, 0)))
    bi = 8
    grid = (s // bi,)
    return pl.pallas_call(
        functools.partial(_body, bi=bi, s=s2, maxrel=maxrel),
        grid=grid,
        in_specs=[
            pl.BlockSpec((bi, s2, d), lambda i: (i, 0, 0)),
            pl.BlockSpec(tpad.shape, lambda i: (0, 0)),
        ],
        out_specs=pl.BlockSpec((bi, s2, d), lambda i: (i, 0, 0)),
        out_shape=jax.ShapeDtypeStruct((s, s2, d), x.dtype),
    )(x, tpad)
